# Initial kernel scaffold; baseline (speedup 1.0000x reference)
import jax
import jax.numpy as jnp
from jax.experimental import pallas as pl


def kernel(x, edge_index, W1l, b1, W1r, W2l, b2, W2r):
    raise NotImplementedError("write your pallas kernel here")


# speedup vs baseline: 1.0232x; 1.0232x over previous
import functools

import jax
import jax.numpy as jnp
from jax.experimental import pallas as pl
from jax.experimental.pallas import tpu as pltpu

_N, _E, _D = 10000, 320000, 128
_BLK = 1000


def _dense_body(relu, aggp_ref, degp_ref, x_ref, Wl_ref, b_ref, Wr_ref, out_ref):
    agg = aggp_ref[0] + aggp_ref[1]
    deg = degp_ref[0][:, 0:1] + degp_ref[1][:, 0:1]
    recip = 1.0 / jnp.maximum(deg, 1.0)
    mean = agg * recip
    y = (jnp.dot(mean, Wl_ref[...], preferred_element_type=jnp.float32)
         + b_ref[...]
         + jnp.dot(x_ref[...], Wr_ref[...], preferred_element_type=jnp.float32))
    out_ref[...] = jnp.maximum(y, 0.0) if relu else y


def _dense_layer(aggp, degp, x, Wl, b, Wr, relu):
    grid = (_N // _BLK,)
    return pl.pallas_call(
        functools.partial(_dense_body, relu),
        grid=grid,
        in_specs=[
            pl.BlockSpec((2, _BLK, _D), lambda i: (0, i, 0)),
            pl.BlockSpec((2, _BLK, 16), lambda i: (0, i, 0)),
            pl.BlockSpec((_BLK, _D), lambda i: (i, 0)),
            pl.BlockSpec((_D, _D), lambda i: (0, 0)),
            pl.BlockSpec((1, _D), lambda i: (0, 0)),
            pl.BlockSpec((_D, _D), lambda i: (0, 0)),
        ],
        out_specs=pl.BlockSpec((_BLK, _D), lambda i: (i, 0)),
        out_shape=jax.ShapeDtypeStruct((_N, _D), jnp.float32),
    )(aggp, degp, x, Wl, b.reshape(1, _D), Wr)


def _agg_partials(x, src, dst):
    agg = jax.ops.segment_sum(jnp.take(x, src, axis=0), dst, num_segments=_N)
    return jnp.stack((agg, jnp.zeros_like(agg)))


def _deg_partials(dst):
    deg = jax.ops.segment_sum(jnp.ones((_E,), jnp.float32), dst, num_segments=_N)
    degp = jnp.broadcast_to(deg[:, None], (_N, 16))
    return jnp.stack((degp, jnp.zeros_like(degp)))


def kernel(x, edge_index, W1l, b1, W1r, W2l, b2, W2r):
    src = edge_index[0]
    dst = edge_index[1]
    degp = _deg_partials(dst)
    agg1 = _agg_partials(x, src, dst)
    h = _dense_layer(agg1, degp, x, W1l, b1, W1r, relu=True)
    agg2 = _agg_partials(h, src, dst)
    out = _dense_layer(agg2, degp, h, W2l, b2, W2r, relu=False)
    return out


# speedup vs baseline: 11.5306x; 11.2688x over previous
import dataclasses
import functools

import jax
import jax.numpy as jnp
from jax import lax
from jax.experimental import pallas as pl
from jax.experimental.pallas import tpu as pltpu
from jax.experimental.pallas import tpu_sc as plsc

_N, _E, _D = 10000, 320000, 128
_BLK = 1000

_NC, _NS = 2, 16
_NW = _NC * _NS
_EW = _E // _NW
_SUB = 125
_JN = 8
_K = _JN * _SUB
_NCH = _EW // _K
_IRW = _EW // _SUB
_NPAD = 10240
_RPS = _NPAD // _NS
_RPS_LAST = _N - (_NS - 1) * _RPS


def _copy_out(src_sp, dst_hbm, c, s):
    out_off = c * _N + s * _RPS

    @pl.when(s < _NS - 1)
    def _():
        pltpu.sync_copy(src_sp.at[pl.ds(s * _RPS, _RPS)],
                        dst_hbm.at[pl.ds(out_off, _RPS)])

    @pl.when(s == _NS - 1)
    def _():
        pltpu.sync_copy(src_sp.at[pl.ds(s * _RPS, _RPS_LAST)],
                        dst_hbm.at[pl.ds(out_off, _RPS_LAST)])


def _sc_agg_body(x_hbm, src_hbm, dst_hbm, z_hbm, agg_out,
                 sidx_v, didx_v, rows_v, acc_sp, sem0, sem1):
    sems = (sem0, sem1)
    c = lax.axis_index("c")
    s = lax.axis_index("s")
    wid = c * _NS + s

    pltpu.sync_copy(z_hbm, acc_sp.at[pl.ds(s * _RPS, _RPS)])
    plsc.subcore_barrier()

    row_base = wid * _IRW

    @pl.loop(0, _NCH)
    def _(t):
        r0 = row_base + t * _JN
        pltpu.sync_copy(src_hbm.at[pl.ds(r0, _JN)], sidx_v)
        pltpu.sync_copy(dst_hbm.at[pl.ds(r0, _JN)], didx_v)
        cps = [None] * _JN
        cps[0] = pltpu.async_copy(x_hbm.at[sidx_v.at[0]], rows_v.at[0], sems[0])
        for j in range(_JN):
            if j + 1 < _JN:
                cps[j + 1] = pltpu.async_copy(
                    x_hbm.at[sidx_v.at[j + 1]], rows_v.at[(j + 1) % 2],
                    sems[(j + 1) % 2])
            cps[j].wait()
            pltpu.sync_copy(rows_v.at[j % 2], acc_sp.at[didx_v.at[j]],
                            add=True)

    plsc.subcore_barrier()
    _copy_out(acc_sp, agg_out, c, s)


_DROWS = _E // _D
_DCHUNKS = _DROWS // _JN
_DTAIL = _DROWS - _DCHUNKS * _JN


def _sc_deg_body(dst_hbm, deg_out, didx_v, acc_v):
    c = lax.axis_index("c")
    s = lax.axis_index("s")
    wid = c * _NS + s

    @pl.loop(0, _N // 16)
    def _(i):
        acc_v[pl.ds(i * 16, 16)] = jnp.zeros((16,), jnp.float32)

    ones16 = jnp.ones((16,), jnp.float32)

    @pl.loop(0, (_DCHUNKS + _NW - 1) // _NW)
    def _(t):
        chunk = wid + t * _NW

        @pl.when(chunk < _DCHUNKS)
        def _():
            pltpu.sync_copy(dst_hbm.at[pl.ds(chunk * _JN, _JN)], didx_v)
            for r in range(_JN):
                for q in range(_D // 16):
                    idx16 = didx_v[r, pl.ds(q * 16, 16)]
                    plsc.addupdate_scatter(acc_v, [idx16], ones16)

    @pl.when(wid == _NW - 1)
    def _():
        pltpu.sync_copy(dst_hbm.at[pl.ds(_DCHUNKS * _JN, _DTAIL)],
                        didx_v.at[pl.ds(0, _DTAIL)])
        for r in range(_DTAIL):
            for q in range(_D // 16):
                idx16 = didx_v[r, pl.ds(q * 16, 16)]
                plsc.addupdate_scatter(acc_v, [idx16], ones16)

    for b in range(_N // _BLK):
        pltpu.sync_copy(acc_v.at[pl.ds(b * _BLK, _BLK)],
                        deg_out.at[pl.ds((b * _NW + wid) * _BLK, _BLK)])


_sc_mesh = plsc.VectorSubcoreMesh(core_axis_name="c", subcore_axis_name="s")

_sc_agg = pl.kernel(
    _sc_agg_body,
    out_type=[jax.ShapeDtypeStruct((2 * _N, _D), jnp.float32)],
    mesh=_sc_mesh,
    scratch_types=[
        pltpu.VMEM((_JN, _SUB), jnp.int32),
        pltpu.VMEM((_JN, _SUB), jnp.int32),
        pltpu.VMEM((2, _SUB, _D), jnp.float32),
        pltpu.VMEM_SHARED((_NPAD, _D), jnp.float32),
        pltpu.SemaphoreType.DMA,
        pltpu.SemaphoreType.DMA,
    ],
)

_sc_deg = pl.kernel(
    _sc_deg_body,
    out_type=[jax.ShapeDtypeStruct((_NW * _N,), jnp.float32)],
    mesh=_sc_mesh,
    scratch_types=[
        pltpu.VMEM((_JN, _D), jnp.int32),
        pltpu.VMEM((_N,), jnp.float32),
    ],
    compiler_params=dataclasses.replace(
        pltpu.CompilerParams(), needs_layout_passes=False),
)


def _dense_body(relu, aggp_ref, degp_ref, x_ref, Wl_ref, b_ref, Wr_ref, out_ref):
    agg = aggp_ref[0] + aggp_ref[1]
    deg = jnp.sum(degp_ref[0], axis=0)[:, None]
    recip = 1.0 / jnp.maximum(deg, 1.0)
    mean = agg * recip
    y = (jnp.dot(mean, Wl_ref[...], preferred_element_type=jnp.float32)
         + b_ref[...]
         + jnp.dot(x_ref[...], Wr_ref[...], preferred_element_type=jnp.float32))
    out_ref[...] = jnp.maximum(y, 0.0) if relu else y


def _dense_layer(aggp, degp, x, Wl, b, Wr, relu):
    grid = (_N // _BLK,)
    return pl.pallas_call(
        functools.partial(_dense_body, relu),
        grid=grid,
        in_specs=[
            pl.BlockSpec((2, _BLK, _D), lambda i: (0, i, 0)),
            pl.BlockSpec((1, _NW, _BLK), lambda i: (i, 0, 0)),
            pl.BlockSpec((_BLK, _D), lambda i: (i, 0)),
            pl.BlockSpec((_D, _D), lambda i: (0, 0)),
            pl.BlockSpec((1, _D), lambda i: (0, 0)),
            pl.BlockSpec((_D, _D), lambda i: (0, 0)),
        ],
        out_specs=pl.BlockSpec((_BLK, _D), lambda i: (i, 0)),
        out_shape=jax.ShapeDtypeStruct((_N, _D), jnp.float32),
    )(aggp, degp, x, Wl, b.reshape(1, _D), Wr)


def kernel(x, edge_index, W1l, b1, W1r, W2l, b2, W2r):
    src2d = edge_index[0].reshape(_E // _SUB, _SUB)
    dst2d = edge_index[1].reshape(_E // _SUB, _SUB)
    dst128 = edge_index[1].reshape(_DROWS, _D)
    zrows = jnp.zeros((_RPS, _D), jnp.float32)

    degp = _sc_deg(dst128)[0].reshape(_N // _BLK, _NW, _BLK)
    agg1 = _sc_agg(x, src2d, dst2d, zrows)[0].reshape(2, _N, _D)
    h = _dense_layer(agg1, degp, x, W1l, b1, W1r, relu=True)
    agg2 = _sc_agg(h, src2d, dst2d, zrows)[0].reshape(2, _N, _D)
    out = _dense_layer(agg2, degp, h, W2l, b2, W2r, relu=False)
    return out
